# Initial kernel scaffold; baseline (speedup 1.0000x reference)
import jax
import jax.numpy as jnp
from jax.experimental import pallas as pl


def kernel(x, edge_index, edge_curvature, Wc1, bc1, Wc2, bc2, W1, b1, W2, b2):
    raise NotImplementedError("write your pallas kernel here")


# speedup vs baseline: 2.6552x; 2.6552x over previous
import functools

import jax
import jax.numpy as jnp
from jax.experimental import pallas as pl

N_NODES = 10000
N_EDGES = 320000


def _mm_kernel(x_ref, w_ref, o_ref):
    o_ref[...] = jnp.dot(x_ref[...], w_ref[...],
                         preferred_element_type=jnp.float32)


def _matmul(x, W):
    M, K = x.shape
    N = W.shape[1]
    BM = 2000
    return pl.pallas_call(
        _mm_kernel,
        grid=(M // BM,),
        in_specs=[pl.BlockSpec((BM, K), lambda i: (i, 0)),
                  pl.BlockSpec((K, N), lambda i: (0, 0))],
        out_specs=pl.BlockSpec((BM, N), lambda i: (i, 0)),
        out_shape=jax.ShapeDtypeStruct((M, N), jnp.float32),
    )(x, W)


def kernel(x, edge_index, edge_curvature, Wc1, bc1, Wc2, bc2, W1, b1, W2, b2):
    src = edge_index[0]
    dst = edge_index[1]
    hcur = jax.nn.relu(edge_curvature @ Wc1 + bc1)
    ew = jax.nn.sigmoid(hcur @ Wc2 + bc2).squeeze(-1)
    ew = 0.1 + 0.9 * ew

    deg = jnp.ones((N_NODES,), jnp.float32).at[dst].add(ew)
    dis = jax.lax.rsqrt(deg)

    def layer(h, W, b):
        g = dis[:, None] * _matmul(h, W)
        msg = g[src] * ew[:, None]
        S = jnp.zeros((N_NODES, g.shape[1]), jnp.float32).at[dst].add(msg)
        return dis[:, None] * (S + g) + b

    h1 = jax.nn.relu(layer(x, W1, b1))
    return layer(h1, W2, b2)


# speedup vs baseline: 2.7549x; 1.0376x over previous
import jax
import jax.numpy as jnp
from jax import lax
from jax.experimental import pallas as pl

N_NODES = 10000
N_EDGES = 320000
D = 128
N2 = 10240
BM = 2048


def _ew_body(c_ref, w1_ref, b1_ref, w2_ref, b2_ref, o_ref):
    c = c_ref[...]
    acc = jnp.zeros_like(c) + b2_ref[0, 0]
    for k in range(16):
        acc += jnp.maximum(c * w1_ref[0, k] + b1_ref[0, k], 0.0) * w2_ref[k, 0]
    o_ref[...] = 0.1 + 0.9 * jax.nn.sigmoid(acc)


def _ew_tc(curv, Wc1, bc1, Wc2, bc2):
    rows = N_EDGES // 128
    return pl.pallas_call(
        _ew_body,
        in_specs=[pl.BlockSpec((rows, 128), lambda: (0, 0)),
                  pl.BlockSpec((1, 16), lambda: (0, 0)),
                  pl.BlockSpec((1, 16), lambda: (0, 0)),
                  pl.BlockSpec((16, 1), lambda: (0, 0)),
                  pl.BlockSpec((1, 1), lambda: (0, 0))],
        out_specs=pl.BlockSpec((rows, 128), lambda: (0, 0)),
        out_shape=jax.ShapeDtypeStruct((rows, 128), jnp.float32),
    )(curv.reshape(rows, 128), Wc1, bc1.reshape(1, 16), Wc2,
      bc2.reshape(1, 1)).reshape(N_EDGES)


def _mm1_body(x_ref, w_ref, deg_ref, g_ref, dis_ref):
    dis = lax.rsqrt(1.0 + deg_ref[0, :])
    h = jnp.dot(x_ref[...], w_ref[...], preferred_element_type=jnp.float32)
    g_ref[...] = h * dis[:, None]
    dis_ref[...] = dis[None, :]


def _mm1(xp, W, deg):
    return pl.pallas_call(
        _mm1_body,
        grid=(N2 // BM,),
        in_specs=[pl.BlockSpec((BM, D), lambda i: (i, 0)),
                  pl.BlockSpec((D, D), lambda i: (0, 0)),
                  pl.BlockSpec((1, BM), lambda i: (0, i))],
        out_specs=[pl.BlockSpec((BM, D), lambda i: (i, 0)),
                   pl.BlockSpec((1, BM), lambda i: (0, i))],
        out_shape=[jax.ShapeDtypeStruct((N2, D), jnp.float32),
                   jax.ShapeDtypeStruct((1, N2), jnp.float32)],
    )(xp, W, deg)


def _mid_body(p_ref, g_ref, dis_ref, b_ref, w_ref, o_ref):
    dis = dis_ref[0, :]
    h = dis[:, None] * (p_ref[...] + g_ref[...]) + b_ref[...]
    h = jnp.maximum(h, 0.0)
    g = jnp.dot(h, w_ref[...], preferred_element_type=jnp.float32)
    o_ref[...] = g * dis[:, None]


def _mid(P, g, dis, b, W):
    return pl.pallas_call(
        _mid_body,
        grid=(N2 // BM,),
        in_specs=[pl.BlockSpec((BM, D), lambda i: (i, 0)),
                  pl.BlockSpec((BM, D), lambda i: (i, 0)),
                  pl.BlockSpec((1, BM), lambda i: (0, i)),
                  pl.BlockSpec((1, D), lambda i: (0, 0)),
                  pl.BlockSpec((D, D), lambda i: (0, 0))],
        out_specs=pl.BlockSpec((BM, D), lambda i: (i, 0)),
        out_shape=jax.ShapeDtypeStruct((N2, D), jnp.float32),
    )(P, g, dis, b.reshape(1, D), W)


def _fin_body(p_ref, g_ref, dis_ref, b_ref, o_ref):
    dis = dis_ref[0, :]
    o_ref[...] = dis[:, None] * (p_ref[...] + g_ref[...]) + b_ref[...]


def _fin(P, g, dis, b):
    return pl.pallas_call(
        _fin_body,
        grid=(N2 // BM,),
        in_specs=[pl.BlockSpec((BM, D), lambda i: (i, 0)),
                  pl.BlockSpec((BM, D), lambda i: (i, 0)),
                  pl.BlockSpec((1, BM), lambda i: (0, i)),
                  pl.BlockSpec((1, D), lambda i: (0, 0))],
        out_specs=pl.BlockSpec((BM, D), lambda i: (i, 0)),
        out_shape=jax.ShapeDtypeStruct((N2, D), jnp.float32),
    )(P, g, dis, b.reshape(1, D))


def kernel(x, edge_index, edge_curvature, Wc1, bc1, Wc2, bc2, W1, b1, W2, b2):
    src = edge_index[0]
    dst = edge_index[1]
    xp = jnp.pad(x, ((0, N2 - N_NODES), (0, 0)))

    ew = _ew_tc(edge_curvature[:, 0], Wc1, bc1, Wc2, bc2)
    deg = jnp.zeros((N2,), jnp.float32).at[dst].add(ew).reshape(1, N2)
    g1, dis = _mm1(xp, W1, deg)

    def segsum(g):
        msg = g[src] * ew[:, None]
        return jnp.zeros((N2, D), jnp.float32).at[dst].add(msg)

    g2 = _mid(segsum(g1), g1, dis, b1, W2)
    out = _fin(segsum(g2), g2, dis, b2)
    return out[:N_NODES]


# speedup vs baseline: 3.0321x; 1.1006x over previous
import functools

import jax
import jax.numpy as jnp
from jax import lax
from jax.experimental import pallas as pl
from jax.experimental.pallas import tpu as pltpu
from jax.experimental.pallas import tpu_sc as plsc

N_NODES = 10000
N_EDGES = 320000
D = 128
N2 = 10240
BM = 2048


def _ew_body(c_ref, w1_ref, b1_ref, w2_ref, b2_ref, o_ref):
    c = c_ref[...]
    acc = jnp.zeros_like(c) + b2_ref[0, 0]
    for k in range(16):
        acc += jnp.maximum(c * w1_ref[0, k] + b1_ref[0, k], 0.0) * w2_ref[k, 0]
    o_ref[...] = 0.1 + 0.9 * jax.nn.sigmoid(acc)


def _ew_tc(curv, Wc1, bc1, Wc2, bc2):
    rows = N_EDGES // 128
    return pl.pallas_call(
        _ew_body,
        in_specs=[pl.BlockSpec((rows, 128), lambda: (0, 0)),
                  pl.BlockSpec((1, 16), lambda: (0, 0)),
                  pl.BlockSpec((1, 16), lambda: (0, 0)),
                  pl.BlockSpec((16, 1), lambda: (0, 0)),
                  pl.BlockSpec((1, 1), lambda: (0, 0))],
        out_specs=pl.BlockSpec((rows, 128), lambda: (0, 0)),
        out_shape=jax.ShapeDtypeStruct((rows, 128), jnp.float32),
    )(curv.reshape(rows, 128), Wc1, bc1.reshape(1, 16), Wc2,
      bc2.reshape(1, 1)).reshape(N_EDGES)


def _mm1_body(x_ref, w_ref, deg_ref, g_ref, dis_ref):
    dis = lax.rsqrt(1.0 + deg_ref[0, :])
    h = jnp.dot(x_ref[...], w_ref[...], preferred_element_type=jnp.float32)
    g_ref[...] = h * dis[:, None]
    dis_ref[...] = dis[None, :]


def _mm1(xp, W, deg):
    return pl.pallas_call(
        _mm1_body,
        grid=(N2 // BM,),
        in_specs=[pl.BlockSpec((BM, D), lambda i: (i, 0)),
                  pl.BlockSpec((D, D), lambda i: (0, 0)),
                  pl.BlockSpec((1, BM), lambda i: (0, i))],
        out_specs=[pl.BlockSpec((BM, D), lambda i: (i, 0)),
                   pl.BlockSpec((1, BM), lambda i: (0, i))],
        out_shape=[jax.ShapeDtypeStruct((N2, D), jnp.float32),
                   jax.ShapeDtypeStruct((1, N2), jnp.float32)],
    )(xp, W, deg)


def _mid_body(p_ref, g_ref, dis_ref, b_ref, w_ref, o_ref):
    dis = dis_ref[0, :]
    h = dis[:, None] * (p_ref[...] + g_ref[...]) + b_ref[...]
    h = jnp.maximum(h, 0.0)
    g = jnp.dot(h, w_ref[...], preferred_element_type=jnp.float32)
    o_ref[...] = g * dis[:, None]


def _mid(P, g, dis, b, W):
    return pl.pallas_call(
        _mid_body,
        grid=(N2 // BM,),
        in_specs=[pl.BlockSpec((BM, D), lambda i: (i, 0)),
                  pl.BlockSpec((BM, D), lambda i: (i, 0)),
                  pl.BlockSpec((1, BM), lambda i: (0, i)),
                  pl.BlockSpec((1, D), lambda i: (0, 0)),
                  pl.BlockSpec((D, D), lambda i: (0, 0))],
        out_specs=pl.BlockSpec((BM, D), lambda i: (i, 0)),
        out_shape=jax.ShapeDtypeStruct((N2, D), jnp.float32),
    )(P, g, dis, b.reshape(1, D), W)


def _fin_body(p_ref, g_ref, dis_ref, b_ref, o_ref):
    dis = dis_ref[0, :]
    o_ref[...] = dis[:, None] * (p_ref[...] + g_ref[...]) + b_ref[...]


def _fin(P, g, dis, b):
    return pl.pallas_call(
        _fin_body,
        grid=(N2 // BM,),
        in_specs=[pl.BlockSpec((BM, D), lambda i: (i, 0)),
                  pl.BlockSpec((BM, D), lambda i: (i, 0)),
                  pl.BlockSpec((1, BM), lambda i: (0, i)),
                  pl.BlockSpec((1, D), lambda i: (0, 0))],
        out_specs=pl.BlockSpec((BM, D), lambda i: (i, 0)),
        out_shape=jax.ShapeDtypeStruct((N2, D), jnp.float32),
    )(P, g, dis, b.reshape(1, D))


NW = 32
EPW = N_EDGES // NW
L = 16
_MESH = plsc.VectorSubcoreMesh(core_axis_name="c", subcore_axis_name="s")
_SC_PARAMS = pltpu.CompilerParams(needs_layout_passes=False)


@functools.partial(
    pl.kernel,
    out_type=jax.ShapeDtypeStruct((NW, N2), jnp.float32),
    mesh=_MESH,
    scratch_types=[pltpu.VMEM((EPW,), jnp.int32),
                   pltpu.VMEM((EPW,), jnp.float32),
                   pltpu.VMEM((N2,), jnp.float32)],
    compiler_params=_SC_PARAMS,
)
def _deg_sc(dstp_hbm, ewp_hbm, out_hbm, dstv, ewv, degv):
    cid = lax.axis_index("c")
    sid = lax.axis_index("s")
    wid = sid * 2 + cid
    pltpu.sync_copy(dstp_hbm.at[wid], dstv)
    pltpu.sync_copy(ewp_hbm.at[wid], ewv)

    @pl.loop(0, N2, step=L)
    def _(i):
        degv[pl.ds(i, L)] = jnp.zeros((L,), jnp.float32)

    @pl.loop(0, EPW, step=L)
    def _(c):
        d = dstv[pl.ds(c, L)]
        w = ewv[pl.ds(c, L)]
        plsc.addupdate_scatter(degv, [d], w)

    pltpu.sync_copy(degv, out_hbm.at[wid])


def kernel(x, edge_index, edge_curvature, Wc1, bc1, Wc2, bc2, W1, b1, W2, b2):
    src = edge_index[0]
    dst = edge_index[1]
    xp = jnp.pad(x, ((0, N2 - N_NODES), (0, 0)))

    ew = _ew_tc(edge_curvature[:, 0], Wc1, bc1, Wc2, bc2)
    degp = _deg_sc(dst.reshape(NW, EPW), ew.reshape(NW, EPW))
    deg = jnp.sum(degp, axis=0).reshape(1, N2)
    g1, dis = _mm1(xp, W1, deg)

    def segsum(g):
        msg = g[src] * ew[:, None]
        return jnp.zeros((N2, D), jnp.float32).at[dst].add(msg)

    g2 = _mid(segsum(g1), g1, dis, b1, W2)
    out = _fin(segsum(g2), g2, dis, b2)
    return out[:N_NODES]
